# baseline (speedup 1.0000x reference)
import functools

import jax
import jax.numpy as jnp
from jax import lax
from jax.experimental import pallas as pl
from jax.experimental.pallas import tpu as pltpu
from jax.experimental.pallas import tpu_sc as plsc

T = 8
D = 2048
OUT = 1024
B = 4096

TILE = 256
MAX_TILES = B // TILE + T - 1
PMAX = MAX_TILES * TILE
NTT = 32

NC, NS = 2, 16
NW = NC * NS
BW = B // NW
CHS = 16
CHG = 32


def _route_body(lab_ref, dst_ref, tt_ref, tv_ref):
    labs = lab_ref[...]
    lower = (lax.broadcasted_iota(jnp.int32, (NW, NW), 1)
             < lax.broadcasted_iota(jnp.int32, (NW, NW), 0)).astype(jnp.float32)
    upper = (lax.broadcasted_iota(jnp.int32, (BW, BW), 0)
             < lax.broadcasted_iota(jnp.int32, (BW, BW), 1)).astype(jnp.float32)

    dstf = jnp.zeros((NW, BW), jnp.float32)
    jlane = lax.broadcasted_iota(jnp.int32, (1, NTT), 1).astype(jnp.float32)
    cnt_tiles = jnp.zeros((1, NTT), jnp.float32)
    pbase = 0.0
    ntiles = 0.0
    for t in range(T):
        oh = (labs == t).astype(jnp.float32)
        cnt = jnp.sum(oh, axis=0, keepdims=True)
        tot = jnp.sum(cnt)
        padded = jnp.ceil(tot / TILE) * TILE
        pref = lax.dot_general(cnt, upper, (((1,), (0,)), ((), ())))
        wtn = lax.dot_general(lower, oh, (((1,), (0,)), ((), ())))
        dstf = dstf + oh * (pbase + pref + wtn)
        etile = (pbase + padded) / TILE
        cnt_tiles = cnt_tiles + (jlane >= etile).astype(jnp.float32)
        pbase = pbase + padded
        ntiles = etile
    dst_ref[...] = dstf.astype(jnp.int32)
    tt_ref[...] = jnp.minimum(cnt_tiles, float(T - 1)).astype(jnp.int32)
    tv_ref[...] = (jlane < ntiles).astype(jnp.int32)


def _scatter_body(x_hbm, dst_hbm, xs_hbm, idx0, idx1, rows0, rows1, rsem, ssem):
    cid = lax.axis_index("c")
    sid = lax.axis_index("s")
    wid = cid * NS + sid
    n = BW // CHS
    idxs = (idx0, idx1)
    bufs = (rows0, rows1)

    def read(c):
        return pltpu.async_copy(
            x_hbm.at[pl.ds(wid * BW + c * CHS, CHS)], bufs[c % 2], rsem)

    reads = [None] * n
    scats = [None] * n
    reads[0] = read(0)
    pltpu.sync_copy(dst_hbm.at[wid, pl.ds(0, CHS)], idxs[0])
    for c in range(n):
        reads[c].wait()
        if c >= 1:
            scats[c - 1].wait()
        if c + 1 < n:
            reads[c + 1] = read(c + 1)
            pltpu.sync_copy(
                dst_hbm.at[wid, pl.ds((c + 1) * CHS, CHS)], idxs[(c + 1) % 2])
        scats[c] = pltpu.async_copy(bufs[c % 2], xs_hbm.at[idxs[c % 2]], ssem)
    scats[n - 1].wait()


def _gather_body(os_hbm, dst_hbm, out_hbm, idxall, buf0, buf1, gsem, wsem):
    cid = lax.axis_index("c")
    sid = lax.axis_index("s")
    wid = cid * NS + sid
    n = BW // CHG
    bufs = (buf0, buf1)
    pltpu.sync_copy(dst_hbm.at[wid], idxall)

    def gath(c):
        return pltpu.async_copy(
            os_hbm.at[idxall.at[pl.ds(c * CHG, CHG)]], bufs[c % 2], gsem)

    gs = [None] * n
    ws = [None] * n
    gs[0] = gath(0)
    for c in range(n):
        gs[c].wait()
        if c >= 1:
            ws[c - 1].wait()
        if c + 1 < n:
            gs[c + 1] = gath(c + 1)
        ws[c] = pltpu.async_copy(
            bufs[c % 2], out_hbm.at[pl.ds(wid * BW + c * CHG, CHG)], wsem)
    ws[n - 1].wait()


def _sc_mesh():
    return plsc.VectorSubcoreMesh(
        core_axis_name="c", subcore_axis_name="s",
        num_cores=NC, num_subcores=NS)


@functools.cache
def _get_scatter():
    return pl.kernel(
        _scatter_body,
        out_type=jax.ShapeDtypeStruct((PMAX, D), jnp.float32),
        mesh=_sc_mesh(),
        scratch_types=[
            pltpu.VMEM((CHS,), jnp.int32),
            pltpu.VMEM((CHS,), jnp.int32),
            pltpu.VMEM((CHS, D), jnp.float32),
            pltpu.VMEM((CHS, D), jnp.float32),
            pltpu.SemaphoreType.DMA,
            pltpu.SemaphoreType.DMA,
        ],
    )


@functools.cache
def _get_gather():
    return pl.kernel(
        _gather_body,
        out_type=jax.ShapeDtypeStruct((B, OUT), jnp.float32),
        mesh=_sc_mesh(),
        scratch_types=[
            pltpu.VMEM((BW,), jnp.int32),
            pltpu.VMEM((CHG, OUT), jnp.float32),
            pltpu.VMEM((CHG, OUT), jnp.float32),
            pltpu.SemaphoreType.DMA,
            pltpu.SemaphoreType.DMA,
        ],
    )


NK = 4
DK = D // NK


def _mm_body(tt_ref, tv_ref, xs_ref, w_ref, b_ref, out_ref):
    i = pl.program_id(0)
    k = pl.program_id(1)

    @pl.when(tv_ref[i] != 0)
    def _():
        acc = lax.dot_general(
            xs_ref[...], w_ref[0],
            dimension_numbers=(((1,), (1,)), ((), ())),
            precision=lax.Precision.DEFAULT,
            preferred_element_type=jnp.float32,
        )

        @pl.when(k == 0)
        def _():
            out_ref[...] = acc + b_ref[0]

        @pl.when(k != 0)
        def _():
            out_ref[...] += acc


def kernel(x, task_labels, W, b):
    labs = task_labels.astype(jnp.int32).reshape(NW, BW)

    dst, tt, tv = pl.pallas_call(
        _route_body,
        out_shape=(
            jax.ShapeDtypeStruct((NW, BW), jnp.int32),
            jax.ShapeDtypeStruct((1, NTT), jnp.int32),
            jax.ShapeDtypeStruct((1, NTT), jnp.int32),
        ),
    )(labs)
    tt1 = tt.reshape(NTT)
    tv1 = tv.reshape(NTT)

    xs = _get_scatter()(x, dst)

    b3 = b.reshape(T, 1, OUT)
    out_sorted = pl.pallas_call(
        _mm_body,
        grid_spec=pltpu.PrefetchScalarGridSpec(
            num_scalar_prefetch=2,
            grid=(MAX_TILES, NK),
            in_specs=[
                pl.BlockSpec((TILE, DK), lambda i, k, tt, tv: (i, k)),
                pl.BlockSpec((1, OUT, DK), lambda i, k, tt, tv: (tt[i], 0, k)),
                pl.BlockSpec((1, 1, OUT), lambda i, k, tt, tv: (tt[i], 0, 0)),
            ],
            out_specs=pl.BlockSpec((TILE, OUT), lambda i, k, tt, tv: (i, 0)),
        ),
        out_shape=jax.ShapeDtypeStruct((PMAX, OUT), jnp.float32),
    )(tt1, tv1, xs, W, b3)

    return _get_gather()(out_sorted, dst)


# speedup vs baseline: 1.5081x; 1.5081x over previous
import functools

import jax
import jax.numpy as jnp
from jax import lax
from jax.experimental import pallas as pl
from jax.experimental.pallas import tpu as pltpu
from jax.experimental.pallas import tpu_sc as plsc

T = 8
D = 2048
OUT = 1024
B = 4096

TILE = 256
MAX_TILES = B // TILE + T - 1
PMAX = MAX_TILES * TILE
NTT = 32

NC, NS = 2, 16
NW = NC * NS
BW = B // NW
CHS = 32
CHG = 64


def _route_body(lab_ref, dst_ref, tt_ref, tv_ref):
    labs = lab_ref[...]
    lower = (lax.broadcasted_iota(jnp.int32, (NW, NW), 1)
             < lax.broadcasted_iota(jnp.int32, (NW, NW), 0)).astype(jnp.float32)
    upper = (lax.broadcasted_iota(jnp.int32, (BW, BW), 0)
             < lax.broadcasted_iota(jnp.int32, (BW, BW), 1)).astype(jnp.float32)

    dstf = jnp.zeros((NW, BW), jnp.float32)
    jlane = lax.broadcasted_iota(jnp.int32, (1, NTT), 1).astype(jnp.float32)
    cnt_tiles = jnp.zeros((1, NTT), jnp.float32)
    pbase = 0.0
    ntiles = 0.0
    for t in range(T):
        oh = (labs == t).astype(jnp.float32)
        cnt = jnp.sum(oh, axis=0, keepdims=True)
        tot = jnp.sum(cnt)
        padded = jnp.ceil(tot / TILE) * TILE
        pref = lax.dot_general(cnt, upper, (((1,), (0,)), ((), ())))
        wtn = lax.dot_general(lower, oh, (((1,), (0,)), ((), ())))
        dstf = dstf + oh * (pbase + pref + wtn)
        etile = (pbase + padded) / TILE
        cnt_tiles = cnt_tiles + (jlane >= etile).astype(jnp.float32)
        pbase = pbase + padded
        ntiles = etile
    dst_ref[...] = dstf.astype(jnp.int32)
    tt_ref[...] = jnp.minimum(cnt_tiles, float(T - 1)).astype(jnp.int32)
    tv_ref[...] = (jlane < ntiles).astype(jnp.int32)


def _scatter_body(x_hbm, dst_hbm, xs_hbm, idxv, rows, sem):
    cid = lax.axis_index("c")
    sid = lax.axis_index("s")
    wid = cid * NS + sid
    for c in range(BW // CHS):
        pltpu.sync_copy(dst_hbm.at[wid, pl.ds(c * CHS, CHS)], idxv)
        pltpu.sync_copy(x_hbm.at[pl.ds(wid * BW + c * CHS, CHS)], rows)
        pltpu.async_copy(rows, xs_hbm.at[idxv], sem).wait()


def _gather_body(os_hbm, dst_hbm, out_hbm, idxv, buf, sem):
    cid = lax.axis_index("c")
    sid = lax.axis_index("s")
    wid = cid * NS + sid
    for c in range(BW // CHG):
        pltpu.sync_copy(dst_hbm.at[wid, pl.ds(c * CHG, CHG)], idxv)
        pltpu.async_copy(os_hbm.at[idxv], buf, sem).wait()
        pltpu.sync_copy(buf, out_hbm.at[pl.ds(wid * BW + c * CHG, CHG)])


def _sc_mesh():
    return plsc.VectorSubcoreMesh(
        core_axis_name="c", subcore_axis_name="s",
        num_cores=NC, num_subcores=NS)


@functools.cache
def _get_scatter():
    return pl.kernel(
        _scatter_body,
        out_type=jax.ShapeDtypeStruct((PMAX, D), jnp.float32),
        mesh=_sc_mesh(),
        scratch_types=[
            pltpu.VMEM((CHS,), jnp.int32),
            pltpu.VMEM((CHS, D), jnp.float32),
            pltpu.SemaphoreType.DMA,
        ],
    )


@functools.cache
def _get_gather():
    return pl.kernel(
        _gather_body,
        out_type=jax.ShapeDtypeStruct((B, OUT), jnp.float32),
        mesh=_sc_mesh(),
        scratch_types=[
            pltpu.VMEM((CHG,), jnp.int32),
            pltpu.VMEM((CHG, OUT), jnp.float32),
            pltpu.SemaphoreType.DMA,
        ],
    )


def _mm_body(tt_ref, tv_ref, xs_ref, w_ref, b_ref, out_ref):
    i = pl.program_id(0)

    @pl.when(tv_ref[i] != 0)
    def _():
        out_ref[...] = lax.dot_general(
            xs_ref[...], w_ref[0],
            dimension_numbers=(((1,), (1,)), ((), ())),
            precision=lax.Precision.DEFAULT,
            preferred_element_type=jnp.float32,
        ) + b_ref[0]


def kernel(x, task_labels, W, b):
    labs = task_labels.astype(jnp.int32).reshape(NW, BW)

    dst, tt, tv = pl.pallas_call(
        _route_body,
        out_shape=(
            jax.ShapeDtypeStruct((NW, BW), jnp.int32),
            jax.ShapeDtypeStruct((1, NTT), jnp.int32),
            jax.ShapeDtypeStruct((1, NTT), jnp.int32),
        ),
    )(labs)
    tt1 = tt.reshape(NTT)
    tv1 = tv.reshape(NTT)

    xs = _get_scatter()(x, dst)

    b3 = b.reshape(T, 1, OUT)
    out_sorted = pl.pallas_call(
        _mm_body,
        grid_spec=pltpu.PrefetchScalarGridSpec(
            num_scalar_prefetch=2,
            grid=(MAX_TILES,),
            in_specs=[
                pl.BlockSpec((TILE, D), lambda i, tt, tv: (i, 0)),
                pl.BlockSpec((1, OUT, D), lambda i, tt, tv: (tt[i], 0, 0)),
                pl.BlockSpec((1, 1, OUT), lambda i, tt, tv: (tt[i], 0, 0)),
            ],
            out_specs=pl.BlockSpec((TILE, OUT), lambda i, tt, tv: (i, 0)),
        ),
        out_shape=jax.ShapeDtypeStruct((PMAX, OUT), jnp.float32),
    )(tt1, tv1, xs, W, b3)

    return _get_gather()(out_sorted, dst)
